# baseline (speedup 1.0000x reference)
import jax
import jax.numpy as jnp
from jax import lax
from jax.experimental import pallas as pl
from jax.experimental.pallas import tpu as pltpu
from jax.experimental.pallas import tpu_sc as plsc

R = 64
C = 1_000_000
B = 26 * 1024
NS = 16
NW = 32
W = 65536
W_LAST = 16896
QW = 16384
TAIL0 = 15 * W + W_LAST
CAP = B + 16
IDX_CH = 1024
LIST_CH = 2048


def _body(x_hbm, xtail_hbm, idx_hbm, out_hbm,
          win2_v, outr_v, outrb_v, cntarr_v, basearr_v, pack_v, idxc_v, stg_v,
          cnt1_v, lists_hbm, cnts_hbm):
    s = lax.axis_index("s")
    cid = lax.axis_index("c")
    wid = cid * NS + s
    iota = lax.iota(jnp.int32, 16)

    pltpu.sync_copy(idx_hbm, outr_v)

    def pass_counts(v, _):
        ivec = outr_v[pl.ds(v * 16, 16)]
        buck = lax.shift_right_logical(ivec, 16)
        msk = buck == s
        c = plsc.all_reduce_population_count(msk)
        plsc.store_scatter(cntarr_v, [iota * 0 + v], c, mask=iota == 0)
        return 0
    lax.fori_loop(0, B // 16, pass_counts, 0)

    def pass_prefix(u, carry):
        c16 = cntarr_v[pl.ds(u * 16, 16)]
        pfx = plsc.cumsum(c16)
        basearr_v[pl.ds(u * 16, 16)] = carry + pfx - c16
        return carry + jnp.max(pfx)
    cnt = lax.fori_loop(0, B // 256, pass_prefix, jnp.int32(0))

    def pass_emit(v, _):
        ivec = outr_v[pl.ds(v * 16, 16)]
        buck = lax.shift_right_logical(ivec, 16)
        msk = buck == s
        off = lax.bitwise_and(ivec, 65535)
        pos = v * 16 + iota
        packed = lax.bitwise_or(lax.shift_left(pos, 16), off)
        mski = jnp.where(msk, 1, 0)
        pfx = plsc.cumsum(mski)
        basev = plsc.load_gather(basearr_v, [iota * 0 + v])
        tpos = basev + pfx - 1
        plsc.store_scatter(pack_v, [tpos], packed, mask=msk)
        return 0
    lax.fori_loop(0, B // 16, pass_emit, 0)

    pltpu.sync_copy(pack_v, lists_hbm.at[wid])
    cnt1_v[pl.ds(0, 16)] = jnp.zeros((16,), jnp.int32) + cnt
    pltpu.sync_copy(cnt1_v.at[pl.ds(0, 16)], cnts_hbm.at[wid])
    plsc.subcore_barrier()

    cnts = []
    for b in range(NS):
        pltpu.sync_copy(cnts_hbm.at[cid * NS + b], cnt1_v.at[pl.ds(0, 16)])
        cnts.append(jnp.max(cnt1_v[pl.ds(0, 16)]))

    r0 = 2 * wid
    zeros16 = iota * 0
    ones16 = zeros16 + 1
    for b in range(NS):
        cnt_b = cnts[b]
        nch = lax.shift_right_logical(cnt_b + (LIST_CH - 1), 11)
        nq = 4 if b < NS - 1 else 2
        for q in range(nq):
            col0 = b * W + q * QW
            if b < NS - 1 or q < 1:
                pltpu.sync_copy(
                    x_hbm.at[pl.ds(r0, 2), pl.ds(col0, QW)], win2_v)
            else:
                pltpu.sync_copy(
                    x_hbm.at[pl.ds(r0, 2), pl.ds(col0, 512)],
                    win2_v.at[:, pl.ds(0, 512)])
                pltpu.sync_copy(xtail_hbm.at[pl.ds(r0, 2)],
                                win2_v.at[:, pl.ds(512, 128)])

            def chunk_body(k, staged):
                @pl.when(k != staged)
                def _():
                    pltpu.sync_copy(
                        lists_hbm.at[cid * NS + b].at[
                            pl.ds(k * LIST_CH, LIST_CH)],
                        stg_v,
                    )
                rem = cnt_b - k * LIST_CH

                def gather_vec(g, _):
                    pk = stg_v[pl.ds(g * 16, 16)]
                    off = lax.bitwise_and(pk, 65535)
                    pos = lax.shift_right_logical(pk, 16)
                    ol = lax.bitwise_and(off, QW - 1)
                    mq = lax.shift_right_logical(off, 14) == q
                    v0 = plsc.bitcast(
                        plsc.load_gather(win2_v, [zeros16, ol]), jnp.int32)
                    v1 = plsc.bitcast(
                        plsc.load_gather(win2_v, [ones16, ol]), jnp.int32)
                    lanemask = lax.bitwise_and((g * 16 + iota) < rem, mq)
                    plsc.store_scatter(outr_v, [pos], v0, mask=lanemask)
                    plsc.store_scatter(outrb_v, [pos], v1, mask=lanemask)
                    return 0
                nv = jnp.minimum(LIST_CH // 16,
                                 lax.shift_right_logical(rem + 15, 4))
                lax.fori_loop(0, nv, gather_vec, 0)
                return k
            lax.fori_loop(0, nch, chunk_body, jnp.int32(-1))
    pltpu.sync_copy(outr_v, out_hbm.at[r0])
    pltpu.sync_copy(outrb_v, out_hbm.at[r0 + 1])


def kernel(x, index):
    idx = index.reshape(B)
    xtail = jnp.pad(x[:, TAIL0:], ((0, 0), (0, 128 - (C - TAIL0))))
    out = pl.kernel(
        _body,
        out_type=jax.ShapeDtypeStruct((R, B), jnp.int32),
        mesh=plsc.VectorSubcoreMesh(core_axis_name="c", subcore_axis_name="s"),
        compiler_params=pltpu.CompilerParams(needs_layout_passes=False),
        scratch_types=[
            pltpu.VMEM((2, QW), jnp.float32),
            pltpu.VMEM((B,), jnp.int32),
            pltpu.VMEM((B,), jnp.int32),
            pltpu.VMEM((B // 16 + 16,), jnp.int32),
            pltpu.VMEM((B // 16 + 16,), jnp.int32),
            pltpu.VMEM((CAP,), jnp.int32),
            pltpu.VMEM((IDX_CH,), jnp.int32),
            pltpu.VMEM((LIST_CH,), jnp.int32),
            pltpu.VMEM((16,), jnp.int32),
            pltpu.HBM((NW, CAP), jnp.int32),
            pltpu.HBM((NW, 16), jnp.int32),
        ],
    )(x, xtail, idx)
    return lax.bitcast_convert_type(out, jnp.float32).reshape(R, 26, 1024)


# speedup vs baseline: 1.3797x; 1.3797x over previous
import jax
import jax.numpy as jnp
from jax import lax
from jax.experimental import pallas as pl
from jax.experimental.pallas import tpu as pltpu
from jax.experimental.pallas import tpu_sc as plsc

R = 64
C = 1_000_000
B = 26 * 1024
NS = 16
NW = 32
W = 65536
W_LAST = 16896
TAIL0 = 15 * W + W_LAST
CAP = B + 16
IDX_CH = 1024
LIST_CH = 2048


def _body(x_hbm, xtail_hbm, idx_hbm, out_hbm,
          win_v, outr_v, cntarr_v, basearr_v, pack_v, idxc_v, stg_v, cnt1_v,
          cntblk_v, sem, lists_hbm, cnts_hbm):
    s = lax.axis_index("s")
    cid = lax.axis_index("c")
    wid = cid * NS + s
    iota = lax.iota(jnp.int32, 16)

    pltpu.sync_copy(idx_hbm, outr_v)

    def pass_counts(v, _):
        ivec = outr_v[pl.ds(v * 16, 16)]
        buck = lax.shift_right_logical(ivec, 16)
        msk = buck == s
        c = plsc.all_reduce_population_count(msk)
        plsc.store_scatter(cntarr_v, [iota * 0 + v], c, mask=iota == 0)
        return 0
    lax.fori_loop(0, B // 16, pass_counts, 0)

    def pass_prefix(u, carry):
        c16 = cntarr_v[pl.ds(u * 16, 16)]
        pfx = plsc.cumsum(c16)
        basearr_v[pl.ds(u * 16, 16)] = carry + pfx - c16
        return carry + jnp.max(pfx)
    cnt = lax.fori_loop(0, B // 256, pass_prefix, jnp.int32(0))

    def pass_emit(v, _):
        ivec = outr_v[pl.ds(v * 16, 16)]
        buck = lax.shift_right_logical(ivec, 16)
        msk = buck == s
        off = lax.bitwise_and(ivec, 65535)
        pos = v * 16 + iota
        packed = lax.bitwise_or(lax.shift_left(pos, 16), off)
        mski = jnp.where(msk, 1, 0)
        pfx = plsc.cumsum(mski)
        basev = plsc.load_gather(basearr_v, [iota * 0 + v])
        tpos = basev + pfx - 1
        plsc.store_scatter(pack_v, [tpos], packed, mask=msk)
        return 0
    lax.fori_loop(0, B // 16, pass_emit, 0)

    pltpu.sync_copy(pack_v, lists_hbm.at[wid])
    cnt1_v[pl.ds(0, 16)] = jnp.zeros((16,), jnp.int32) + cnt
    pltpu.sync_copy(cnt1_v.at[pl.ds(0, 16)], cnts_hbm.at[wid])
    plsc.subcore_barrier()

    pltpu.sync_copy(cnts_hbm.at[pl.ds(cid * NS, NS)], cntblk_v)
    cnts = [jnp.max(cntblk_v[b, pl.ds(0, 16)]) for b in range(NS)]

    def do_row(i):
        for b in range(NS):
            if b < NS - 1:
                cps = [
                    pltpu.async_copy(
                        x_hbm.at[i].at[pl.ds(b * W + h * (W // 4), W // 4)],
                        win_v.at[pl.ds(h * (W // 4), W // 4)],
                        sem,
                    )
                    for h in range(4)
                ]
                for cp in cps:
                    cp.wait()
            else:
                pltpu.sync_copy(
                    x_hbm.at[i].at[pl.ds(b * W, W_LAST)],
                    win_v.at[pl.ds(0, W_LAST)],
                )
                pltpu.sync_copy(xtail_hbm.at[i], win_v.at[pl.ds(W_LAST, 128)])
            cnt_b = cnts[b]

            def chunk_body(k, _):
                pltpu.sync_copy(
                    lists_hbm.at[cid * NS + b].at[pl.ds(k * LIST_CH, LIST_CH)],
                    stg_v,
                )
                rem = cnt_b - k * LIST_CH

                def gather_vec(g, _):
                    pk = stg_v[pl.ds(g * 16, 16)]
                    off = lax.bitwise_and(pk, 65535)
                    pos = lax.shift_right_logical(pk, 16)
                    vals = plsc.bitcast(plsc.load_gather(win_v, [off]),
                                        jnp.int32)
                    lanemask = (g * 16 + iota) < rem
                    plsc.store_scatter(outr_v, [pos], vals, mask=lanemask)
                    return 0
                nv = jnp.minimum(LIST_CH // 16,
                                 lax.shift_right_logical(rem + 15, 4))
                lax.fori_loop(0, nv, gather_vec, 0)
                return 0
            nch = lax.shift_right_logical(cnt_b + (LIST_CH - 1), 11)
            lax.fori_loop(0, nch, chunk_body, 0)
        pltpu.sync_copy(outr_v, out_hbm.at[i])

    do_row(2 * wid)
    do_row(2 * wid + 1)


def kernel(x, index):
    idx = index.reshape(B)
    xtail = jnp.pad(x[:, TAIL0:], ((0, 0), (0, 128 - (C - TAIL0))))
    out = pl.kernel(
        _body,
        out_type=jax.ShapeDtypeStruct((R, B), jnp.int32),
        mesh=plsc.VectorSubcoreMesh(core_axis_name="c", subcore_axis_name="s"),
        compiler_params=pltpu.CompilerParams(needs_layout_passes=False),
        scratch_types=[
            pltpu.VMEM((W,), jnp.float32),
            pltpu.VMEM((B,), jnp.int32),
            pltpu.VMEM((B // 16 + 16,), jnp.int32),
            pltpu.VMEM((B // 16 + 16,), jnp.int32),
            pltpu.VMEM((CAP,), jnp.int32),
            pltpu.VMEM((IDX_CH,), jnp.int32),
            pltpu.VMEM((LIST_CH,), jnp.int32),
            pltpu.VMEM((16,), jnp.int32),
            pltpu.VMEM((NS, 16), jnp.int32),
            pltpu.SemaphoreType.DMA,
            pltpu.HBM((NW, CAP), jnp.int32),
            pltpu.HBM((NW, 16), jnp.int32),
        ],
    )(x, xtail, idx)
    return lax.bitcast_convert_type(out, jnp.float32).reshape(R, 26, 1024)


# speedup vs baseline: 1.4900x; 1.0799x over previous
import jax
import jax.numpy as jnp
from jax import lax
from jax.experimental import pallas as pl
from jax.experimental.pallas import tpu as pltpu
from jax.experimental.pallas import tpu_sc as plsc

R = 64
C = 1_000_000
B = 26 * 1024
NS = 16
NW = 32
W = 65536
W_LAST = 16896
TAIL0 = 15 * W + W_LAST
CAP = B + 16
IDX_CH = 1024
LIST_CH = 2048


def _body(x_hbm, xtail_hbm, idx_hbm, out_hbm,
          win_v, outr_v, cntarr_v, basearr_v, pack_v, idxc_v, stg_v, cnt1_v,
          cntblk_v, sem, lists_hbm, cnts_hbm):
    s = lax.axis_index("s")
    cid = lax.axis_index("c")
    wid = cid * NS + s
    iota = lax.iota(jnp.int32, 16)

    pltpu.sync_copy(idx_hbm, outr_v)

    def pass_counts(v4, _):
        for u in range(4):
            v = v4 * 4 + u
            ivec = outr_v[pl.ds(v * 16, 16)]
            buck = lax.shift_right_logical(ivec, 16)
            msk = buck == s
            c = plsc.all_reduce_population_count(msk)
            plsc.store_scatter(cntarr_v, [iota * 0 + v], c, mask=iota == 0)
        return 0
    lax.fori_loop(0, B // 64, pass_counts, 0)

    def pass_prefix(u, carry):
        c16 = cntarr_v[pl.ds(u * 16, 16)]
        pfx = plsc.cumsum(c16)
        basearr_v[pl.ds(u * 16, 16)] = carry + pfx - c16
        return carry + jnp.max(pfx)
    cnt = lax.fori_loop(0, B // 256, pass_prefix, jnp.int32(0))

    def pass_emit(v4, _):
        for u in range(4):
            v = v4 * 4 + u
            ivec = outr_v[pl.ds(v * 16, 16)]
            buck = lax.shift_right_logical(ivec, 16)
            msk = buck == s
            off = lax.bitwise_and(ivec, 65535)
            pos = v * 16 + iota
            packed = lax.bitwise_or(lax.shift_left(pos, 16), off)
            mski = jnp.where(msk, 1, 0)
            pfx = plsc.cumsum(mski)
            basev = plsc.load_gather(basearr_v, [iota * 0 + v])
            tpos = basev + pfx - 1
            plsc.store_scatter(pack_v, [tpos], packed, mask=msk)
        return 0
    lax.fori_loop(0, B // 64, pass_emit, 0)

    def pub_chunk(k, _):
        pltpu.sync_copy(pack_v.at[pl.ds(k * LIST_CH, LIST_CH)],
                        lists_hbm.at[wid].at[pl.ds(k * LIST_CH, LIST_CH)])
        return 0
    lax.fori_loop(0, lax.shift_right_logical(cnt + (LIST_CH - 1), 11),
                  pub_chunk, 0)
    cnt1_v[pl.ds(0, 16)] = jnp.zeros((16,), jnp.int32) + cnt
    pltpu.sync_copy(cnt1_v.at[pl.ds(0, 16)], cnts_hbm.at[wid])
    plsc.subcore_barrier()

    pltpu.sync_copy(cnts_hbm.at[pl.ds(cid * NS, NS)], cntblk_v)
    cnts = [jnp.max(cntblk_v[b, pl.ds(0, 16)]) for b in range(NS)]

    def do_row(i):
        for b in range(NS):
            cp0 = pltpu.async_copy(
                lists_hbm.at[cid * NS + b].at[pl.ds(0, LIST_CH)], stg_v, sem)
            if b < NS - 1:
                cps = [
                    pltpu.async_copy(
                        x_hbm.at[i].at[pl.ds(b * W + h * (W // 4), W // 4)],
                        win_v.at[pl.ds(h * (W // 4), W // 4)],
                        sem,
                    )
                    for h in range(4)
                ]
                for cp in cps:
                    cp.wait()
            else:
                pltpu.sync_copy(
                    x_hbm.at[i].at[pl.ds(b * W, W_LAST)],
                    win_v.at[pl.ds(0, W_LAST)],
                )
                pltpu.sync_copy(xtail_hbm.at[i], win_v.at[pl.ds(W_LAST, 128)])
            cp0.wait()
            cnt_b = cnts[b]

            def chunk_body(k, _):
                @pl.when(k != 0)
                def _():
                    pltpu.sync_copy(
                        lists_hbm.at[cid * NS + b].at[
                            pl.ds(k * LIST_CH, LIST_CH)],
                        stg_v,
                    )
                rem = cnt_b - k * LIST_CH

                def gather_vec(g, _):
                    pk = stg_v[pl.ds(g * 16, 16)]
                    off = lax.bitwise_and(pk, 65535)
                    pos = lax.shift_right_logical(pk, 16)
                    vals = plsc.bitcast(plsc.load_gather(win_v, [off]),
                                        jnp.int32)
                    lanemask = (g * 16 + iota) < rem
                    plsc.store_scatter(outr_v, [pos], vals, mask=lanemask)
                    return 0
                nv = jnp.minimum(LIST_CH // 16,
                                 lax.shift_right_logical(rem + 15, 4))
                lax.fori_loop(0, nv, gather_vec, 0)
                return 0
            nch = lax.shift_right_logical(cnt_b + (LIST_CH - 1), 11)
            lax.fori_loop(0, nch, chunk_body, 0)
        pltpu.sync_copy(outr_v, out_hbm.at[i])

    do_row(2 * wid)
    do_row(2 * wid + 1)


def kernel(x, index):
    idx = index.reshape(B)
    xtail = jnp.pad(x[:, TAIL0:], ((0, 0), (0, 128 - (C - TAIL0))))
    out = pl.kernel(
        _body,
        out_type=jax.ShapeDtypeStruct((R, B), jnp.int32),
        mesh=plsc.VectorSubcoreMesh(core_axis_name="c", subcore_axis_name="s"),
        compiler_params=pltpu.CompilerParams(needs_layout_passes=False),
        scratch_types=[
            pltpu.VMEM((W,), jnp.float32),
            pltpu.VMEM((B,), jnp.int32),
            pltpu.VMEM((B // 16 + 16,), jnp.int32),
            pltpu.VMEM((B // 16 + 16,), jnp.int32),
            pltpu.VMEM((CAP,), jnp.int32),
            pltpu.VMEM((IDX_CH,), jnp.int32),
            pltpu.VMEM((LIST_CH,), jnp.int32),
            pltpu.VMEM((16,), jnp.int32),
            pltpu.VMEM((NS, 16), jnp.int32),
            pltpu.SemaphoreType.DMA,
            pltpu.HBM((NW, CAP), jnp.int32),
            pltpu.HBM((NW, 16), jnp.int32),
        ],
    )(x, xtail, idx)
    return lax.bitcast_convert_type(out, jnp.float32).reshape(R, 26, 1024)


# speedup vs baseline: 1.4927x; 1.0018x over previous
import jax
import jax.numpy as jnp
from jax import lax
from jax.experimental import pallas as pl
from jax.experimental.pallas import tpu as pltpu
from jax.experimental.pallas import tpu_sc as plsc

R = 64
C = 1_000_000
B = 26 * 1024
NS = 16
NW = 32
W = 65536
W_LAST = 16896
TAIL0 = 15 * W + W_LAST
CAP = B + 16
IDX_CH = 1024
LIST_CH = 2048


def _body(x_hbm, xtail_hbm, idx_hbm, out_hbm,
          win_v, outr_v, cntarr_v, basearr_v, pack_v, idxc_v, stg_v, cnt1_v,
          cntblk_v, sem, lists_hbm, cnts_hbm):
    s = lax.axis_index("s")
    cid = lax.axis_index("c")
    wid = cid * NS + s
    iota = lax.iota(jnp.int32, 16)

    pltpu.sync_copy(idx_hbm, outr_v)

    def pass_counts(v4, _):
        for u in range(4):
            v = v4 * 4 + u
            ivec = outr_v[pl.ds(v * 16, 16)]
            buck = lax.shift_right_logical(ivec, 16)
            msk = buck == s
            c = plsc.all_reduce_population_count(msk)
            plsc.store_scatter(cntarr_v, [iota * 0 + v], c, mask=iota == 0)
        return 0
    lax.fori_loop(0, B // 64, pass_counts, 0)

    def pass_prefix(u, carry):
        c16 = cntarr_v[pl.ds(u * 16, 16)]
        pfx = plsc.cumsum(c16)
        basearr_v[pl.ds(u * 16, 16)] = carry + pfx - c16
        return carry + jnp.max(pfx)
    cnt = lax.fori_loop(0, B // 256, pass_prefix, jnp.int32(0))

    def pass_emit(v4, _):
        for u in range(4):
            v = v4 * 4 + u
            ivec = outr_v[pl.ds(v * 16, 16)]
            buck = lax.shift_right_logical(ivec, 16)
            msk = buck == s
            off = lax.bitwise_and(ivec, 65535)
            pos = v * 16 + iota
            packed = lax.bitwise_or(lax.shift_left(pos, 16), off)
            mski = jnp.where(msk, 1, 0)
            pfx = plsc.cumsum(mski)
            basev = plsc.load_gather(basearr_v, [iota * 0 + v])
            tpos = basev + pfx - 1
            plsc.store_scatter(pack_v, [tpos], packed, mask=msk)
        return 0
    lax.fori_loop(0, B // 64, pass_emit, 0)

    def pub_chunk(k, _):
        pltpu.sync_copy(pack_v.at[pl.ds(k * LIST_CH, LIST_CH)],
                        lists_hbm.at[wid].at[pl.ds(k * LIST_CH, LIST_CH)])
        return 0
    lax.fori_loop(0, lax.shift_right_logical(cnt + (LIST_CH - 1), 11),
                  pub_chunk, 0)
    cnt1_v[pl.ds(0, 16)] = jnp.zeros((16,), jnp.int32) + cnt
    pltpu.sync_copy(cnt1_v.at[pl.ds(0, 16)], cnts_hbm.at[wid])
    plsc.subcore_barrier()

    pltpu.sync_copy(cnts_hbm.at[pl.ds(cid * NS, NS)], cntblk_v)
    cnts = [jnp.max(cntblk_v[b, pl.ds(0, 16)]) for b in range(NS)]

    def do_row(i):
        for b in range(NS):
            cp0 = pltpu.async_copy(
                lists_hbm.at[cid * NS + b].at[pl.ds(0, LIST_CH)], stg_v, sem)
            if b < NS - 1:
                cps = [
                    pltpu.async_copy(
                        x_hbm.at[i].at[pl.ds(b * W + h * (W // 8), W // 8)],
                        win_v.at[pl.ds(h * (W // 8), W // 8)],
                        sem,
                    )
                    for h in range(8)
                ]
                for cp in cps:
                    cp.wait()
            else:
                pltpu.sync_copy(
                    x_hbm.at[i].at[pl.ds(b * W, W_LAST)],
                    win_v.at[pl.ds(0, W_LAST)],
                )
                pltpu.sync_copy(xtail_hbm.at[i], win_v.at[pl.ds(W_LAST, 128)])
            cp0.wait()
            cnt_b = cnts[b]

            def chunk_body(k, _):
                @pl.when(k != 0)
                def _():
                    pltpu.sync_copy(
                        lists_hbm.at[cid * NS + b].at[
                            pl.ds(k * LIST_CH, LIST_CH)],
                        stg_v,
                    )
                rem = cnt_b - k * LIST_CH

                def gather_vec(g2, _):
                    for u in range(2):
                        g = g2 * 2 + u
                        pk = stg_v[pl.ds(g * 16, 16)]
                        off = lax.bitwise_and(pk, 65535)
                        pos = lax.shift_right_logical(pk, 16)
                        vals = plsc.bitcast(plsc.load_gather(win_v, [off]),
                                            jnp.int32)
                        lanemask = (g * 16 + iota) < rem
                        plsc.store_scatter(outr_v, [pos], vals, mask=lanemask)
                    return 0
                nv = jnp.minimum(LIST_CH // 32,
                                 lax.shift_right_logical(rem + 31, 5))
                lax.fori_loop(0, nv, gather_vec, 0)
                return 0
            nch = lax.shift_right_logical(cnt_b + (LIST_CH - 1), 11)
            lax.fori_loop(0, nch, chunk_body, 0)
        pltpu.sync_copy(outr_v, out_hbm.at[i])

    do_row(2 * wid)
    do_row(2 * wid + 1)


def kernel(x, index):
    idx = index.reshape(B)
    xtail = jnp.pad(x[:, TAIL0:], ((0, 0), (0, 128 - (C - TAIL0))))
    out = pl.kernel(
        _body,
        out_type=jax.ShapeDtypeStruct((R, B), jnp.int32),
        mesh=plsc.VectorSubcoreMesh(core_axis_name="c", subcore_axis_name="s"),
        compiler_params=pltpu.CompilerParams(needs_layout_passes=False),
        scratch_types=[
            pltpu.VMEM((W,), jnp.float32),
            pltpu.VMEM((B,), jnp.int32),
            pltpu.VMEM((B // 16 + 16,), jnp.int32),
            pltpu.VMEM((B // 16 + 16,), jnp.int32),
            pltpu.VMEM((CAP,), jnp.int32),
            pltpu.VMEM((IDX_CH,), jnp.int32),
            pltpu.VMEM((LIST_CH,), jnp.int32),
            pltpu.VMEM((16,), jnp.int32),
            pltpu.VMEM((NS, 16), jnp.int32),
            pltpu.SemaphoreType.DMA,
            pltpu.HBM((NW, CAP), jnp.int32),
            pltpu.HBM((NW, 16), jnp.int32),
        ],
    )(x, xtail, idx)
    return lax.bitcast_convert_type(out, jnp.float32).reshape(R, 26, 1024)
